# Initial kernel scaffold; baseline (speedup 1.0000x reference)
import jax
import jax.numpy as jnp
from jax.experimental import pallas as pl


def kernel(x, Wr, br, We, be):
    raise NotImplementedError("write your pallas kernel here")


# speedup vs baseline: 2.2286x; 2.2286x over previous
import functools

import jax
import jax.numpy as jnp
from jax.experimental import pallas as pl


def _moe_body(x_ref, wrt_ref, br_ref, wet_ref, be_ref, o_ref, *, tile, E, C):
    xt = x_ref[...]
    xb = xt.astype(jnp.bfloat16)
    logits = jax.lax.dot_general(
        xb, wrt_ref[...].astype(jnp.bfloat16), (((1,), (0,)), ((), ())),
        preferred_element_type=jnp.float32,
    ) + br_ref[...]

    iota = jax.lax.broadcasted_iota(jnp.int32, (tile, E), 1)
    m1 = jnp.max(logits, axis=1, keepdims=True)
    i1 = jnp.min(jnp.where(logits == m1, iota, E), axis=1, keepdims=True)
    rem = jnp.where(iota == i1, -jnp.inf, logits)
    m2 = jnp.max(rem, axis=1, keepdims=True)
    i2 = jnp.min(jnp.where(rem == m2, iota, E), axis=1, keepdims=True)
    w1 = 1.0 / (1.0 + jnp.exp(m2 - m1))
    w2 = 1.0 - w1

    acc = jnp.zeros((tile, C), jnp.float32)
    for e in range(E):
        coef = jnp.where(i1 == e, w1, 0.0) + jnp.where(i2 == e, w2, 0.0)
        y = jax.lax.dot_general(
            xb, wet_ref[e], (((1,), (0,)), ((), ())),
            preferred_element_type=jnp.float32,
        )
        acc = acc + coef * (y + be_ref[e : e + 1, :])
    o_ref[...] = acc


def kernel(x, Wr, br, We, be):
    B, N, D = x.shape
    E, C, _ = We.shape
    T = B * N
    tile = 256

    xf = x.reshape(T, D)
    wrt = Wr.T
    br2 = br.reshape(1, E)
    wet = jnp.swapaxes(We, 1, 2).astype(jnp.bfloat16)

    out = pl.pallas_call(
        functools.partial(_moe_body, tile=tile, E=E, C=C),
        grid=(T // tile,),
        in_specs=[
            pl.BlockSpec((tile, D), lambda i: (i, 0)),
            pl.BlockSpec((D, E), lambda i: (0, 0)),
            pl.BlockSpec((1, E), lambda i: (0, 0)),
            pl.BlockSpec((E, D, C), lambda i: (0, 0, 0)),
            pl.BlockSpec((E, C), lambda i: (0, 0)),
        ],
        out_specs=pl.BlockSpec((tile, C), lambda i: (i, 0)),
        out_shape=jax.ShapeDtypeStruct((T, C), jnp.float32),
    )(xf, wrt, br2, wet, be)
    return out.reshape(B, N, C)
